# baseline (speedup 1.0000x reference)
import functools

import jax
import jax.numpy as jnp
from jax import lax
from jax.experimental import pallas as pl
from jax.experimental.pallas import tpu as pltpu
from jax.experimental.pallas import tpu_sc as plsc

NUM_CORES = 2
NUM_SUBCORES = 16
NW = NUM_CORES * NUM_SUBCORES
CHUNK = 128
DP = 128


def _gather(idx3d, tabp):
    _, chunks_per_w, _ = idx3d.shape
    B = NW * chunks_per_w * CHUNK
    per_w = B // NW
    mesh = plsc.VectorSubcoreMesh(core_axis_name="c", subcore_axis_name="s")

    @functools.partial(
        pl.kernel,
        out_type=jax.ShapeDtypeStruct((B, DP), jnp.float32),
        mesh=mesh,
        scratch_types=[
            pltpu.VMEM((chunks_per_w, CHUNK), jnp.int32),
            pltpu.VMEM((CHUNK, DP), jnp.float32),
            pltpu.VMEM((CHUNK, DP), jnp.float32),
            pltpu.SemaphoreType.DMA,
            pltpu.SemaphoreType.DMA,
        ],
    )
    def k(idx_hbm, tab_hbm, out_hbm, idx_v, rows0, rows1, sem0, sem1):
        wid = lax.axis_index("s") * NUM_CORES + lax.axis_index("c")
        pltpu.sync_copy(idx_hbm.at[wid], idx_v)
        obase = wid * per_w
        bufs = (rows0, rows1)
        sems = (sem0, sem1)

        pltpu.async_copy(tab_hbm.at[idx_v.at[0]], rows0, sem0)

        @pl.loop(0, chunks_per_w, step=2)
        def chunk_loop(j):
            for b in range(2):
                jj = j + b

                @pl.when(jj + 1 < chunks_per_w)
                def _():
                    pltpu.async_copy(
                        tab_hbm.at[idx_v.at[jj + 1]], bufs[1 - b], sems[1 - b]
                    )

                pltpu.make_async_copy(
                    tab_hbm.at[idx_v.at[jj]], bufs[b], sems[b]
                ).wait()
                pltpu.sync_copy(
                    bufs[b], out_hbm.at[pl.ds(obase + jj * CHUNK, CHUNK)]
                )

    return k(idx3d, tabp)


def kernel(x, table):
    B = x.size
    D = table.shape[1]
    idx3d = x.reshape(NW, B // (NW * CHUNK), CHUNK).astype(jnp.int32)
    tabp = jnp.pad(table, ((0, 0), (0, DP - D)))
    out = _gather(idx3d, tabp)
    return out[:, :D].reshape(x.shape + (D,))
